# baseline (speedup 1.0000x reference)
import functools
import math

import jax
import jax.numpy as jnp
from jax import lax
from jax.experimental import pallas as pl
from jax.experimental.pallas import tpu as pltpu

_DECAY = 0.99
_LR = 0.01
_CHUNK = 1024


def _fwd_kernel(x_ref, wkvq_ref, bkvq_ref, wo_ref, bo_ref, y_ref, state_ref,
                *, C, M, ln_decay):
    @pl.when(pl.program_id(1) == 0)
    def _():
        state_ref[...] = jnp.zeros_like(state_ref)

    xc = x_ref[0].astype(jnp.bfloat16)
    c11 = (((1,), (1,)), ((), ()))
    kvq = lax.dot_general(xc, wkvq_ref[...], c11,
                          preferred_element_type=jnp.float32) + bkvq_ref[...]
    k = kvq[:, :M]
    v = kvq[:, M:2 * M]
    q = kvq[:, 2 * M:]
    kb = k.astype(jnp.bfloat16)
    vb = v.astype(jnp.bfloat16)
    qb = q.astype(jnp.bfloat16)

    state = state_ref[0]

    i_cm = lax.broadcasted_iota(jnp.int32, (C, M), 0).astype(jnp.float32)
    inter = lax.dot_general(qb, state.astype(jnp.bfloat16), c11,
                            preferred_element_type=jnp.float32)
    inter = inter * jnp.exp(i_cm * ln_decay)

    ii = lax.broadcasted_iota(jnp.int32, (C, C), 0).astype(jnp.float32)
    jj = lax.broadcasted_iota(jnp.int32, (C, C), 1).astype(jnp.float32)
    mask = jnp.where(jj < ii, jnp.exp((ii - 1.0 - jj) * ln_decay), 0.0)
    a = lax.dot_general(qb, kb, c11, preferred_element_type=jnp.float32) * mask
    intra = lax.dot_general(a.astype(jnp.bfloat16), vb,
                            (((1,), (0,)), ((), ())),
                            preferred_element_type=jnp.float32)

    outs = inter + _LR * intra
    y_ref[0] = lax.dot_general(outs.astype(jnp.bfloat16), wo_ref[...], c11,
                               preferred_element_type=jnp.float32) + bo_ref[...]

    w = jnp.exp((C - 1.0 - i_cm) * ln_decay)
    supd = lax.dot_general((v * w).astype(jnp.bfloat16), kb,
                           (((0,), (0,)), ((), ())),
                           preferred_element_type=jnp.float32)
    state_ref[0] = (_DECAY ** C) * state + _LR * supd


def kernel(x, Wk, bk, Wv, bv, Wq, bq, Wo, bo):
    B, S, D = x.shape
    M = Wk.shape[0]
    C = _CHUNK
    assert S % C == 0
    wkvq = jnp.concatenate([Wk, Wv, Wq], axis=0).astype(jnp.bfloat16)
    bkvq = jnp.concatenate([bk, bv, bq], axis=0).reshape(1, 3 * M)
    body = functools.partial(_fwd_kernel, C=C, M=M, ln_decay=math.log(_DECAY))
    y, state = pl.pallas_call(
        body,
        grid=(B, S // C),
        in_specs=[
            pl.BlockSpec((1, C, D), lambda b, c: (b, c, 0)),
            pl.BlockSpec((3 * M, D), lambda b, c: (0, 0)),
            pl.BlockSpec((1, 3 * M), lambda b, c: (0, 0)),
            pl.BlockSpec((D, M), lambda b, c: (0, 0)),
            pl.BlockSpec((1, D), lambda b, c: (0, 0)),
        ],
        out_specs=[
            pl.BlockSpec((1, C, D), lambda b, c: (b, c, 0)),
            pl.BlockSpec((1, M, M), lambda b, c: (b, 0, 0)),
        ],
        out_shape=[
            jax.ShapeDtypeStruct((B, S, D), jnp.float32),
            jax.ShapeDtypeStruct((B, M, M), jnp.float32),
        ],
        compiler_params=pltpu.CompilerParams(
            dimension_semantics=("parallel", "arbitrary"),
        ),
    )(x, wkvq, bkvq, Wo.astype(jnp.bfloat16), bo.reshape(1, D))
    return (y, state)


# speedup vs baseline: 1.0359x; 1.0359x over previous
import functools
import math

import jax
import jax.numpy as jnp
from jax import lax
from jax.experimental import pallas as pl
from jax.experimental.pallas import tpu as pltpu

_DECAY = 0.99
_LR = 0.01
_CHUNK = 512


def _fwd_kernel(x_ref, wkvq_ref, bkvq_ref, wo_ref, bo_ref, y_ref, state_ref,
                *, C, M, ln_decay):
    @pl.when(pl.program_id(1) == 0)
    def _():
        state_ref[...] = jnp.zeros_like(state_ref)

    xc = x_ref[0].astype(jnp.bfloat16)
    c11 = (((1,), (1,)), ((), ()))
    kvq = lax.dot_general(xc, wkvq_ref[...], c11,
                          preferred_element_type=jnp.float32) + bkvq_ref[...]
    k = kvq[:, :M]
    v = kvq[:, M:2 * M]
    q = kvq[:, 2 * M:]
    kb = k.astype(jnp.bfloat16)
    vb = v.astype(jnp.bfloat16)
    qb = q.astype(jnp.bfloat16)

    state = state_ref[0]

    i_cm = lax.broadcasted_iota(jnp.int32, (C, M), 0).astype(jnp.float32)
    inter = lax.dot_general(qb, state.astype(jnp.bfloat16), c11,
                            preferred_element_type=jnp.float32)
    inter = inter * jnp.exp(i_cm * ln_decay)

    ii = lax.broadcasted_iota(jnp.int32, (C, C), 0).astype(jnp.float32)
    jj = lax.broadcasted_iota(jnp.int32, (C, C), 1).astype(jnp.float32)
    mask = jnp.where(jj < ii, jnp.exp((ii - 1.0 - jj) * ln_decay), 0.0)
    a = lax.dot_general(qb, kb, c11, preferred_element_type=jnp.float32) * mask
    intra = lax.dot_general(a.astype(jnp.bfloat16), vb,
                            (((1,), (0,)), ((), ())),
                            preferred_element_type=jnp.float32)

    outs = inter + _LR * intra
    y_ref[0] = lax.dot_general(outs.astype(jnp.bfloat16), wo_ref[...], c11,
                               preferred_element_type=jnp.float32) + bo_ref[...]

    w = jnp.exp((C - 1.0 - i_cm) * ln_decay)
    supd = lax.dot_general((v * w).astype(jnp.bfloat16), kb,
                           (((0,), (0,)), ((), ())),
                           preferred_element_type=jnp.float32)
    state_ref[0] = (_DECAY ** C) * state + _LR * supd


def kernel(x, Wk, bk, Wv, bv, Wq, bq, Wo, bo):
    B, S, D = x.shape
    M = Wk.shape[0]
    C = _CHUNK
    assert S % C == 0
    wkvq = jnp.concatenate([Wk, Wv, Wq], axis=0).astype(jnp.bfloat16)
    bkvq = jnp.concatenate([bk, bv, bq], axis=0).reshape(1, 3 * M)
    body = functools.partial(_fwd_kernel, C=C, M=M, ln_decay=math.log(_DECAY))
    y, state = pl.pallas_call(
        body,
        grid=(B, S // C),
        in_specs=[
            pl.BlockSpec((1, C, D), lambda b, c: (b, c, 0)),
            pl.BlockSpec((3 * M, D), lambda b, c: (0, 0)),
            pl.BlockSpec((1, 3 * M), lambda b, c: (0, 0)),
            pl.BlockSpec((D, M), lambda b, c: (0, 0)),
            pl.BlockSpec((1, D), lambda b, c: (0, 0)),
        ],
        out_specs=[
            pl.BlockSpec((1, C, D), lambda b, c: (b, c, 0)),
            pl.BlockSpec((1, M, M), lambda b, c: (b, 0, 0)),
        ],
        out_shape=[
            jax.ShapeDtypeStruct((B, S, D), jnp.float32),
            jax.ShapeDtypeStruct((B, M, M), jnp.float32),
        ],
        compiler_params=pltpu.CompilerParams(
            dimension_semantics=("arbitrary", "arbitrary"),
        ),
    )(x, wkvq, bkvq, Wo.astype(jnp.bfloat16), bo.reshape(1, D))
    return (y, state)
